# baseline (speedup 1.0000x reference)
import jax
import jax.numpy as jnp
from jax import lax
from jax.experimental import pallas as pl
from jax.experimental.pallas import tpu as pltpu
from jax.experimental.pallas import tpu_sc as plsc

PATCH = 196
EMBED = 768
BATCH = 256
LANES = 16
MPAD = 208
NCHUNK = MPAD // LANES
NRING = 4
NC = 2
NS = 16
NW = NC * NS
B_PER_W = BATCH // NW
ROWS = BATCH * PATCH


def _sc_body(x_hbm, maskp_hbm, zeros_hbm, out_hbm,
             mask_v, keep_v, zero_v, gbuf, zbuf,
             gsems, ssems, zsem):
    wid = lax.axis_index("s") * NC + lax.axis_index("c")

    pltpu.sync_copy(maskp_hbm, mask_v)
    pltpu.sync_copy(zeros_hbm, zbuf)

    ones = jnp.full((LANES,), 1, jnp.int32)
    zeros16 = jnp.full((LANES,), 0, jnp.int32)
    nk_v = zeros16
    nz_v = zeros16
    lastk = jnp.int32(0)
    for j in range(NCHUNK):
        m = mask_v[pl.ds(j * LANES, LANES)]
        p = j * LANES + lax.iota(jnp.int32, LANES)
        valid = p < PATCH
        keep = (m == 0) & (p > 0) & valid
        zero = jnp.logical_not(keep) & valid
        kpos = jnp.maximum(nk_v + lax.cumsum(jnp.where(keep, ones, zeros16)) - 1, 0)
        zpos = jnp.maximum(nz_v + lax.cumsum(jnp.where(zero, ones, zeros16)) - 1, 0)
        plsc.store_scatter(keep_v, [kpos], p, mask=keep)
        plsc.store_scatter(zero_v, [zpos], p, mask=zero)
        nk_v = nk_v + plsc.all_reduce_population_count(keep)
        nz_v = nz_v + plsc.all_reduce_population_count(zero)
        lastk = jnp.maximum(lastk, jnp.max(jnp.where(keep, p, zeros16)))
    lastk_v = jnp.full((LANES,), 1, jnp.int32) * lastk
    for j in range(NCHUNK):
        p = j * LANES + lax.iota(jnp.int32, LANES)
        plsc.store_scatter(keep_v, [p], lastk_v, mask=p >= nk_v)
        plsc.store_scatter(zero_v, [p], zeros16, mask=p >= nz_v)
    n_keep = jnp.max(nk_v)
    n_zero = jnp.max(nz_v)

    def kidx(j, basev):
        del basev
        return keep_v[pl.ds(j * LANES, LANES)]

    def zidx(j, basev):
        del basev
        return zero_v[pl.ds(j * LANES, LANES)]

    def batch_body(bl, carry):
        b = wid * B_PER_W + bl
        basev = None
        x_b = x_hbm.at[b]
        out_b = out_hbm.at[b]

        for j in range(NCHUNK):

            @pl.when(j * LANES < n_zero)
            def _():
                pltpu.async_copy(zbuf, out_b.at[zidx(j, basev)], zsem)

        for j in range(NCHUNK + 1):
            if j < NCHUNK:
                s = j % NRING

                @pl.when(j * LANES < n_keep)
                def _():
                    if j >= NRING:
                        pltpu.make_async_copy(
                            gbuf.at[s], out_b.at[kidx(j - NRING, basev)],
                            ssems.at[s]).wait()
                    pltpu.async_copy(x_b.at[kidx(j, basev)], gbuf.at[s],
                                     gsems.at[s])

            if j >= 1:
                jj = j - 1
                s = jj % NRING

                @pl.when(jj * LANES < n_keep)
                def _():
                    pltpu.make_async_copy(x_b.at[kidx(jj, basev)],
                                          gbuf.at[s], gsems.at[s]).wait()
                    pltpu.async_copy(gbuf.at[s], out_b.at[kidx(jj, basev)],
                                     ssems.at[s])

        for j in range(NCHUNK):
            s = j % NRING

            @pl.when((j * LANES < n_keep) & ((j + NRING) * LANES >= n_keep))
            def _():
                pltpu.make_async_copy(gbuf.at[s],
                                      out_b.at[kidx(j, basev)],
                                      ssems.at[s]).wait()

        for j in range(NCHUNK):

            @pl.when(j * LANES < n_zero)
            def _():
                pltpu.make_async_copy(zbuf, out_b.at[zidx(j, basev)],
                                      zsem).wait()

        return carry

    lax.fori_loop(0, B_PER_W, batch_body, 0)


def kernel(x, mask):
    maskp = jnp.concatenate(
        [mask.reshape(-1), jnp.ones((MPAD - PATCH,), mask.dtype)]
    )
    zeros = jnp.zeros((LANES, EMBED), x.dtype)
    mesh = plsc.VectorSubcoreMesh(core_axis_name="c", subcore_axis_name="s")
    out = pl.kernel(
        _sc_body,
        out_type=jax.ShapeDtypeStruct(x.shape, x.dtype),
        mesh=mesh,
        compiler_params=pltpu.CompilerParams(
            needs_layout_passes=False, use_tc_tiling_on_sc=True),
        scratch_types=[
            pltpu.VMEM((MPAD,), jnp.int32),
            pltpu.VMEM((MPAD,), jnp.int32),
            pltpu.VMEM((MPAD,), jnp.int32),
            pltpu.VMEM((NRING, LANES, EMBED), jnp.float32),
            pltpu.VMEM((LANES, EMBED), jnp.float32),
            pltpu.SemaphoreType.DMA((NRING,)),
            pltpu.SemaphoreType.DMA((NRING,)),
            pltpu.SemaphoreType.DMA,
        ],
    )(x, maskp, zeros)
    return (out, mask)


# speedup vs baseline: 2.8848x; 2.8848x over previous
import jax
import jax.numpy as jnp
from jax import lax
from jax.experimental import pallas as pl
from jax.experimental.pallas import tpu as pltpu
from jax.experimental.pallas import tpu_sc as plsc

PATCH = 196
EMBED = 768
BATCH = 256
LANES = 16
MPAD = 208
NCHUNK = MPAD // LANES
NRING = 4
NC = 2
NS = 16
NW = NC * NS
B_PER_W = BATCH // NW
ROWS = BATCH * PATCH


def _sc_body(x_hbm, maskp_hbm, zeros_hbm, out_hbm,
             mask_v, keep_v, zero_v, gbuf, zbuf,
             gsems, ssems, zsem):
    wid = lax.axis_index("s") * NC + lax.axis_index("c")

    pltpu.sync_copy(maskp_hbm, mask_v)
    pltpu.sync_copy(zeros_hbm, zbuf)

    ones = jnp.full((LANES,), 1, jnp.int32)
    zeros16 = jnp.full((LANES,), 0, jnp.int32)
    nk_v = zeros16
    nz_v = zeros16
    lastk = jnp.int32(0)
    for j in range(NCHUNK):
        m = mask_v[pl.ds(j * LANES, LANES)]
        p = j * LANES + lax.iota(jnp.int32, LANES)
        valid = p < PATCH
        keep = (m == 0) & (p > 0) & valid
        zero = jnp.logical_not(keep) & valid
        kpos = jnp.maximum(nk_v + lax.cumsum(jnp.where(keep, ones, zeros16)) - 1, 0)
        zpos = jnp.maximum(nz_v + lax.cumsum(jnp.where(zero, ones, zeros16)) - 1, 0)
        plsc.store_scatter(keep_v, [kpos], p, mask=keep)
        plsc.store_scatter(zero_v, [zpos], p, mask=zero)
        nk_v = nk_v + plsc.all_reduce_population_count(keep)
        nz_v = nz_v + plsc.all_reduce_population_count(zero)
        lastk = jnp.maximum(lastk, jnp.max(jnp.where(keep, p, zeros16)))
    lastk_v = jnp.full((LANES,), 1, jnp.int32) * lastk
    for j in range(NCHUNK):
        p = j * LANES + lax.iota(jnp.int32, LANES)
        plsc.store_scatter(keep_v, [p], lastk_v, mask=p >= nk_v)
        plsc.store_scatter(zero_v, [p], zeros16, mask=p >= nz_v)
    n_keep = jnp.max(nk_v)
    n_zero = jnp.max(nz_v)

    def kidx(j, basev):
        return keep_v[pl.ds(j * LANES, LANES)] * BATCH + basev

    def zidx(j, basev):
        return zero_v[pl.ds(j * LANES, LANES)] * BATCH + basev

    def batch_body(bl, carry):
        b = wid * B_PER_W + bl
        basev = jnp.full((LANES,), b, jnp.int32)
        x_b = x_hbm
        out_b = out_hbm

        for j in range(NCHUNK):

            @pl.when(j * LANES < n_zero)
            def _():
                pltpu.async_copy(zbuf, out_b.at[zidx(j, basev)], zsem)

        for j in range(NCHUNK + 1):
            if j < NCHUNK:
                s = j % NRING

                @pl.when(j * LANES < n_keep)
                def _():
                    if j >= NRING:
                        pltpu.make_async_copy(
                            gbuf.at[s], out_b.at[kidx(j - NRING, basev)],
                            ssems.at[s]).wait()
                    pltpu.async_copy(x_b.at[kidx(j, basev)], gbuf.at[s],
                                     gsems.at[s])

            if j >= 1:
                jj = j - 1
                s = jj % NRING

                @pl.when(jj * LANES < n_keep)
                def _():
                    pltpu.make_async_copy(x_b.at[kidx(jj, basev)],
                                          gbuf.at[s], gsems.at[s]).wait()
                    pltpu.async_copy(gbuf.at[s], out_b.at[kidx(jj, basev)],
                                     ssems.at[s])

        for j in range(NCHUNK):
            s = j % NRING

            @pl.when((j * LANES < n_keep) & ((j + NRING) * LANES >= n_keep))
            def _():
                pltpu.make_async_copy(gbuf.at[s],
                                      out_b.at[kidx(j, basev)],
                                      ssems.at[s]).wait()

        for j in range(NCHUNK):

            @pl.when(j * LANES < n_zero)
            def _():
                pltpu.make_async_copy(zbuf, out_b.at[zidx(j, basev)],
                                      zsem).wait()

        return carry

    lax.fori_loop(0, B_PER_W, batch_body, 0)


def kernel(x, mask):
    xt = jnp.transpose(x, (1, 0, 2)).reshape(ROWS, EMBED)
    maskp = jnp.concatenate(
        [mask.reshape(-1), jnp.ones((MPAD - PATCH,), mask.dtype)]
    )
    zeros = jnp.zeros((LANES, EMBED), x.dtype)
    mesh = plsc.VectorSubcoreMesh(core_axis_name="c", subcore_axis_name="s")
    out = pl.kernel(
        _sc_body,
        out_type=jax.ShapeDtypeStruct((ROWS, EMBED), x.dtype),
        mesh=mesh,
        compiler_params=pltpu.CompilerParams(needs_layout_passes=False),
        scratch_types=[
            pltpu.VMEM((MPAD,), jnp.int32),
            pltpu.VMEM((MPAD,), jnp.int32),
            pltpu.VMEM((MPAD,), jnp.int32),
            pltpu.VMEM((NRING, LANES, EMBED), jnp.float32),
            pltpu.VMEM((LANES, EMBED), jnp.float32),
            pltpu.SemaphoreType.DMA((NRING,)),
            pltpu.SemaphoreType.DMA((NRING,)),
            pltpu.SemaphoreType.DMA,
        ],
    )(xt, maskp, zeros)
    out = jnp.transpose(out.reshape(PATCH, BATCH, EMBED), (1, 0, 2))
    return (out, mask)
